# baseline (speedup 1.0000x reference)
import functools

import jax
import jax.numpy as jnp
from jax import lax
from jax.experimental import pallas as pl
from jax.experimental.pallas import tpu as pltpu
from jax.experimental.pallas import tpu_sc as plsc

H = 768
F_DIM = 768
F2 = 2 * F_DIM
E = 8
T = 2048
BT = 256
NB = T // BT
BF = 128
NF = T // BF
LOG_BF = 7
STEPS = NF + E - 1
TW = 32
NC = 2
NS = 16
NW = NC * NS
CHUNK = T // NW


def _erf(v):
    p = 0.3275911
    a1, a2, a3, a4, a5 = (0.254829592, -0.284496736, 1.421413741,
                          -1.453152027, 1.061405429)
    sg = jnp.sign(v)
    av = jnp.abs(v)
    t = 1.0 / (1.0 + p * av)
    poly = ((((a5 * t + a4) * t + a3) * t + a2) * t + a1) * t
    return sg * (1.0 - poly * jnp.exp(-av * av))


def _gelu(v):
    return 0.5 * v * (1.0 + _erf(v * 0.7071067811865476))


def _incl_scan(v, width):
    def _sh(u, k):
        return jnp.concatenate(
            [jnp.zeros((1, k), jnp.float32), u[:, :width - k]], axis=1)

    k = 1
    while k < width:
        v = v + _sh(v, k)
        k *= 2
    return v


def _lanes_to_sublanes(v):
    d = (lax.broadcasted_iota(jnp.int32, (E, E), 0)
         == lax.broadcasted_iota(jnp.int32, (E, E), 1))
    return jnp.sum(jnp.where(d, v, 0.0), axis=1, keepdims=True)


def _router_body(gate_w_ref, gate_b_ref, x_ref, pos_ref, loss_ref, tbl_ref,
                 idx_scr, rank_scr, cnt_scr):
    s = pl.program_id(0)

    @pl.when(s == 0)
    def _():
        cnt_scr[...] = jnp.zeros_like(cnt_scr)

    @pl.when(s < NB)
    def _():
        x = x_ref[...]
        raw = lax.dot_general(x, gate_w_ref[...], (((1,), (1,)), ((), ())),
                              preferred_element_type=jnp.float32)
        raw = raw + gate_b_ref[...]
        eids = lax.broadcasted_iota(jnp.int32, (BT, E), 1)
        m = jnp.max(raw, axis=1, keepdims=True)
        idx = jnp.min(jnp.where(raw == m, eids, E), axis=1)
        onehot = (eids == idx[:, None]).astype(jnp.bfloat16)

        rit = lax.broadcasted_iota(jnp.int32, (BT, BT), 0)
        cit = lax.broadcasted_iota(jnp.int32, (BT, BT), 1)
        ltri = (rit > cit).astype(jnp.bfloat16)
        within = lax.dot_general(ltri, onehot, (((1,), (0,)), ((), ())),
                                 preferred_element_type=jnp.float32)
        rank_mat = within + cnt_scr[...]
        rank = jnp.sum(jnp.where(eids == idx[:, None], rank_mat, 0.0), axis=1)

        idx_scr[pl.ds(s * BT, BT), :] = idx[:, None]
        rank_scr[pl.ds(s * BT, BT), :] = rank.astype(jnp.int32)[:, None]
        cnt_scr[...] = cnt_scr[...] + jnp.sum(
            onehot.astype(jnp.float32), axis=0, keepdims=True)

    @pl.when(s == NB)
    def _():
        cnt = cnt_scr[...]
        usage = (cnt > 0.0).astype(jnp.float32)
        loss_ref[...] = (jnp.sum((usage - 1.0 / E) ** 2) + 1e-8).reshape(1, 1)

        cw = jnp.concatenate([cnt, jnp.zeros((1, TW - E), jnp.float32)],
                             axis=1)
        incl = _incl_scan(cw, TW)
        offs_row = incl - cw

        idx_all = idx_scr[...]
        rank_all = rank_scr[...]
        eids_t = lax.broadcasted_iota(jnp.int32, (T, E), 1)
        offs_e = offs_row[:, :E]
        picked = jnp.sum(jnp.where(eids_t == idx_all, offs_e, 0.0),
                         axis=1, keepdims=True)
        pos_ref[...] = rank_all + picked.astype(jnp.int32)

        lanew = lax.broadcasted_iota(jnp.int32, (1, TW), 1)
        offs_i = offs_row.astype(jnp.int32)
        c_i = cw.astype(jnp.int32)
        blk_start = lax.shift_right_logical(offs_i, LOG_BF)
        blk_end = jnp.where(c_i > 0,
                            lax.shift_right_logical(offs_i + c_i - 1, LOG_BF),
                            blk_start - 1)
        nblk = jnp.maximum(blk_end - blk_start + 1, 0)
        nblk = jnp.where(lanew < E, nblk, 0).astype(jnp.float32)
        cumnb = _incl_scan(nblk, TW)
        cumnb_excl = cumnb - nblk
        tp = jnp.max(cumnb)

        cumnb_sub = _lanes_to_sublanes(cumnb[:, :E])
        base_sub = _lanes_to_sublanes(
            (blk_start.astype(jnp.float32) - cumnb_excl)[:, :E])
        sw = lanew.astype(jnp.float32)
        eid = jnp.sum((cumnb_sub <= sw).astype(jnp.float32),
                      axis=0, keepdims=True)
        eid = jnp.minimum(eid, float(E - 1))
        erow = lax.broadcasted_iota(jnp.int32, (E, TW), 0).astype(jnp.float32)
        base = jnp.sum(jnp.where(erow == eid, base_sub, 0.0),
                       axis=0, keepdims=True)
        valid = (sw < tp).astype(jnp.float32)
        bid = jnp.where(valid > 0, base + sw, float(NF - 1))
        prev = jnp.concatenate(
            [jnp.full((1, 1), -1.0, jnp.float32), bid[:, :TW - 1]], axis=1)
        first = ((lanew == 0) | (bid != prev)).astype(jnp.float32)

        z = jnp.zeros((1, TW), jnp.float32)
        tbl = jnp.concatenate(
            [bid, eid, first, valid, offs_row, z, z, z], axis=0)
        tbl_ref[...] = tbl.astype(jnp.int32)


def _router_call(x, gate_w, gate_b):
    return pl.pallas_call(
        _router_body,
        grid=(NB + 1,),
        in_specs=[
            pl.BlockSpec((E, H), lambda s: (0, 0)),
            pl.BlockSpec((1, E), lambda s: (0, 0)),
            pl.BlockSpec((BT, H), lambda s: (s % NB, 0)),
        ],
        out_specs=[
            pl.BlockSpec((T, 1), lambda s: (0, 0)),
            pl.BlockSpec((1, 1), lambda s: (0, 0)),
            pl.BlockSpec((E, TW), lambda s: (0, 0)),
        ],
        out_shape=[
            jax.ShapeDtypeStruct((T, 1), jnp.int32),
            jax.ShapeDtypeStruct((1, 1), jnp.float32),
            jax.ShapeDtypeStruct((E, TW), jnp.int32),
        ],
        scratch_shapes=[pltpu.VMEM((T, 1), jnp.int32),
                        pltpu.VMEM((T, 1), jnp.int32),
                        pltpu.VMEM((1, E), jnp.float32)],
    )(gate_w, gate_b.reshape(1, E), x)


@functools.cache
def _get_dispatch():
    mesh = plsc.VectorSubcoreMesh(core_axis_name="c", subcore_axis_name="s")

    @functools.partial(
        pl.kernel,
        mesh=mesh,
        out_type=jax.ShapeDtypeStruct((T, H), jnp.float32),
        scratch_types=[
            pltpu.VMEM((CHUNK,), jnp.int32),
            pltpu.VMEM((CHUNK, H), jnp.float32),
            pltpu.SemaphoreType.DMA,
        ],
    )
    def _dispatch(x_hbm, pos_hbm, xs_hbm, pos_v, x_v, sem):
        wid = lax.axis_index("s") * NC + lax.axis_index("c")
        base = wid * CHUNK
        pltpu.sync_copy(pos_hbm.at[pl.ds(base, CHUNK)], pos_v)
        pltpu.sync_copy(x_hbm.at[pl.ds(base, CHUNK)], x_v)
        pltpu.async_copy(x_v, xs_hbm.at[pos_v], sem).wait()

    return _dispatch


@functools.cache
def _get_combine():
    mesh = plsc.VectorSubcoreMesh(core_axis_name="c", subcore_axis_name="s")

    @functools.partial(
        pl.kernel,
        mesh=mesh,
        out_type=jax.ShapeDtypeStruct((T, H), jnp.float32),
        scratch_types=[
            pltpu.VMEM((CHUNK,), jnp.int32),
            pltpu.VMEM((CHUNK, H), jnp.float32),
            pltpu.SemaphoreType.DMA,
        ],
    )
    def _combine(ys_hbm, pos_hbm, out_hbm, pos_v, y_v, sem):
        wid = lax.axis_index("s") * NC + lax.axis_index("c")
        base = wid * CHUNK
        pltpu.sync_copy(pos_hbm.at[pl.ds(base, CHUNK)], pos_v)
        pltpu.async_copy(ys_hbm.at[pos_v], y_v, sem).wait()
        pltpu.sync_copy(y_v, out_hbm.at[pl.ds(base, CHUNK)])

    return _combine


def _ffn_body(tbl_ref, xs_ref, fcw_ref, fcb_ref, outw_ref, outb_ref, ys_ref):
    s = pl.program_id(0)
    e = tbl_ref[1, s]
    b = tbl_ref[0, s]
    x = xs_ref[...].astype(jnp.bfloat16)
    h = lax.dot_general(x, fcw_ref[0].astype(jnp.bfloat16),
                        (((1,), (1,)), ((), ())),
                        preferred_element_type=jnp.float32)
    h = h + fcb_ref[0]
    g = h[:, :F_DIM] * _gelu(h[:, F_DIM:])
    eo = lax.dot_general(g.astype(jnp.bfloat16),
                         outw_ref[0].astype(jnp.bfloat16),
                         (((1,), (1,)), ((), ())),
                         preferred_element_type=jnp.float32)
    eo = eo + outb_ref[0]
    r = b * BF + lax.broadcasted_iota(jnp.int32, (BF, 1), 0)
    keep = (r >= tbl_ref[4, e]) & (r < tbl_ref[4, e + 1]) & (tbl_ref[3, s] > 0)
    contrib = jnp.where(keep, eo, 0.0)

    @pl.when(tbl_ref[2, s] == 1)
    def _():
        ys_ref[...] = contrib

    @pl.when(tbl_ref[2, s] != 1)
    def _():
        ys_ref[...] = ys_ref[...] + contrib


def _ffn_call(tbl, xs, fc_w, fc_b, out_w, out_b):
    grid_spec = pltpu.PrefetchScalarGridSpec(
        num_scalar_prefetch=1,
        grid=(STEPS,),
        in_specs=[
            pl.BlockSpec((BF, H), lambda s, tbl: (tbl[0, s], 0)),
            pl.BlockSpec((1, F2, H), lambda s, tbl: (tbl[1, s], 0, 0)),
            pl.BlockSpec((1, 1, F2), lambda s, tbl: (tbl[1, s], 0, 0)),
            pl.BlockSpec((1, H, F_DIM), lambda s, tbl: (tbl[1, s], 0, 0)),
            pl.BlockSpec((1, 1, H), lambda s, tbl: (tbl[1, s], 0, 0)),
        ],
        out_specs=pl.BlockSpec((BF, H), lambda s, tbl: (tbl[0, s], 0)),
    )
    return pl.pallas_call(
        _ffn_body,
        grid_spec=grid_spec,
        out_shape=jax.ShapeDtypeStruct((T, H), jnp.float32),
    )(tbl, xs, fc_w, fc_b.reshape(E, 1, F2), out_w, out_b.reshape(E, 1, H))


def kernel(x, gate_w, gate_b, fc_w, fc_b, out_w, out_b):
    pos2, loss11, tbl = _router_call(x, gate_w, gate_b)
    pos = pos2.reshape(T)
    xs = _get_dispatch()(x, pos)
    ys = _ffn_call(tbl, xs, fc_w, fc_b, out_w, out_b)
    out = _get_combine()(ys, pos)
    return out, loss11.reshape(())


# speedup vs baseline: 1.1542x; 1.1542x over previous
import functools

import jax
import jax.numpy as jnp
from jax import lax
from jax.experimental import pallas as pl
from jax.experimental.pallas import tpu as pltpu
from jax.experimental.pallas import tpu_sc as plsc

H = 768
F_DIM = 768
F2 = 2 * F_DIM
E = 8
T = 2048
BT = 256
NB = T // BT
BF = 256
NF = T // BF
LOG_BF = 8
STEPS = NF + E - 1
TW = 32
NC = 2
NS = 16
NW = NC * NS
CHUNK = T // NW


def _erf(v):
    p = 0.3275911
    a1, a2, a3, a4, a5 = (0.254829592, -0.284496736, 1.421413741,
                          -1.453152027, 1.061405429)
    sg = jnp.sign(v)
    av = jnp.abs(v)
    t = 1.0 / (1.0 + p * av)
    poly = ((((a5 * t + a4) * t + a3) * t + a2) * t + a1) * t
    return sg * (1.0 - poly * jnp.exp(-av * av))


def _gelu(v):
    return 0.5 * v * (1.0 + _erf(v * 0.7071067811865476))


def _incl_scan(v, width):
    def _sh(u, k):
        return jnp.concatenate(
            [jnp.zeros((1, k), jnp.float32), u[:, :width - k]], axis=1)

    k = 1
    while k < width:
        v = v + _sh(v, k)
        k *= 2
    return v


def _lanes_to_sublanes(v):
    d = (lax.broadcasted_iota(jnp.int32, (E, E), 0)
         == lax.broadcasted_iota(jnp.int32, (E, E), 1))
    return jnp.sum(jnp.where(d, v, 0.0), axis=1, keepdims=True)


def _router_body(gate_w_ref, gate_b_ref, x_ref, pos_ref, loss_ref, tbl_ref,
                 idx_scr, rank_scr, cnt_scr):
    s = pl.program_id(0)

    @pl.when(s == 0)
    def _():
        cnt_scr[...] = jnp.zeros_like(cnt_scr)

    @pl.when(s < NB)
    def _():
        x = x_ref[...]
        raw = lax.dot_general(x, gate_w_ref[...], (((1,), (1,)), ((), ())),
                              preferred_element_type=jnp.float32)
        raw = raw + gate_b_ref[...]
        eids = lax.broadcasted_iota(jnp.int32, (BT, E), 1)
        m = jnp.max(raw, axis=1, keepdims=True)
        idx = jnp.min(jnp.where(raw == m, eids, E), axis=1)
        onehot = (eids == idx[:, None]).astype(jnp.bfloat16)

        rit = lax.broadcasted_iota(jnp.int32, (BT, BT), 0)
        cit = lax.broadcasted_iota(jnp.int32, (BT, BT), 1)
        ltri = (rit > cit).astype(jnp.bfloat16)
        within = lax.dot_general(ltri, onehot, (((1,), (0,)), ((), ())),
                                 preferred_element_type=jnp.float32)
        rank_mat = within + cnt_scr[...]
        rank = jnp.sum(jnp.where(eids == idx[:, None], rank_mat, 0.0), axis=1)

        idx_scr[pl.ds(s * BT, BT), :] = idx[:, None]
        rank_scr[pl.ds(s * BT, BT), :] = rank.astype(jnp.int32)[:, None]
        cnt_scr[...] = cnt_scr[...] + jnp.sum(
            onehot.astype(jnp.float32), axis=0, keepdims=True)

    @pl.when(s == NB)
    def _():
        cnt = cnt_scr[...]
        usage = (cnt > 0.0).astype(jnp.float32)
        loss_ref[...] = (jnp.sum((usage - 1.0 / E) ** 2) + 1e-8).reshape(1, 1)

        cw = jnp.concatenate([cnt, jnp.zeros((1, TW - E), jnp.float32)],
                             axis=1)
        incl = _incl_scan(cw, TW)
        offs_row = incl - cw

        idx_all = idx_scr[...]
        rank_all = rank_scr[...]
        eids_t = lax.broadcasted_iota(jnp.int32, (T, E), 1)
        offs_e = offs_row[:, :E]
        picked = jnp.sum(jnp.where(eids_t == idx_all, offs_e, 0.0),
                         axis=1, keepdims=True)
        pos = rank_all + picked.astype(jnp.int32)
        pos_ref[...] = pos.reshape(T // 128, 128)

        lanew = lax.broadcasted_iota(jnp.int32, (1, TW), 1)
        offs_i = offs_row.astype(jnp.int32)
        c_i = cw.astype(jnp.int32)
        blk_start = lax.shift_right_logical(offs_i, LOG_BF)
        blk_end = jnp.where(c_i > 0,
                            lax.shift_right_logical(offs_i + c_i - 1, LOG_BF),
                            blk_start - 1)
        nblk = jnp.maximum(blk_end - blk_start + 1, 0)
        nblk = jnp.where(lanew < E, nblk, 0).astype(jnp.float32)
        cumnb = _incl_scan(nblk, TW)
        cumnb_excl = cumnb - nblk
        tp = jnp.max(cumnb)

        cumnb_sub = _lanes_to_sublanes(cumnb[:, :E])
        base_sub = _lanes_to_sublanes(
            (blk_start.astype(jnp.float32) - cumnb_excl)[:, :E])
        sw = lanew.astype(jnp.float32)
        eid = jnp.sum((cumnb_sub <= sw).astype(jnp.float32),
                      axis=0, keepdims=True)
        eid = jnp.minimum(eid, float(E - 1))
        erow = lax.broadcasted_iota(jnp.int32, (E, TW), 0).astype(jnp.float32)
        base = jnp.sum(jnp.where(erow == eid, base_sub, 0.0),
                       axis=0, keepdims=True)
        valid = (sw < tp).astype(jnp.float32)
        bid = jnp.where(valid > 0, base + sw, float(NF - 1))
        prev = jnp.concatenate(
            [jnp.full((1, 1), -1.0, jnp.float32), bid[:, :TW - 1]], axis=1)
        first = ((lanew == 0) | (bid != prev)).astype(jnp.float32)

        z = jnp.zeros((1, TW), jnp.float32)
        tbl = jnp.concatenate(
            [bid, eid, first, valid, offs_row, z, z, z], axis=0)
        tbl_ref[...] = tbl.astype(jnp.int32)


def _router_call(x, gate_w, gate_b):
    return pl.pallas_call(
        _router_body,
        grid=(NB + 1,),
        in_specs=[
            pl.BlockSpec((E, H), lambda s: (0, 0)),
            pl.BlockSpec((1, E), lambda s: (0, 0)),
            pl.BlockSpec((BT, H), lambda s: (s % NB, 0)),
        ],
        out_specs=[
            pl.BlockSpec((T // 128, 128), lambda s: (0, 0)),
            pl.BlockSpec((1, 1), lambda s: (0, 0)),
            pl.BlockSpec((E, TW), lambda s: (0, 0)),
        ],
        out_shape=[
            jax.ShapeDtypeStruct((T // 128, 128), jnp.int32),
            jax.ShapeDtypeStruct((1, 1), jnp.float32),
            jax.ShapeDtypeStruct((E, TW), jnp.int32),
        ],
        scratch_shapes=[pltpu.VMEM((T, 1), jnp.int32),
                        pltpu.VMEM((T, 1), jnp.int32),
                        pltpu.VMEM((1, E), jnp.float32)],
    )(gate_w, gate_b.reshape(1, E), x)


@functools.cache
def _get_dispatch():
    mesh = plsc.VectorSubcoreMesh(core_axis_name="c", subcore_axis_name="s")

    @functools.partial(
        pl.kernel,
        mesh=mesh,
        out_type=jax.ShapeDtypeStruct((T, H), jnp.float32),
        scratch_types=[
            pltpu.VMEM((CHUNK,), jnp.int32),
            pltpu.VMEM((CHUNK, H), jnp.float32),
            pltpu.SemaphoreType.DMA,
        ],
    )
    def _dispatch(x_hbm, pos_hbm, xs_hbm, pos_v, x_v, sem):
        wid = lax.axis_index("s") * NC + lax.axis_index("c")
        base = wid * CHUNK
        pltpu.sync_copy(pos_hbm.at[pl.ds(base, CHUNK)], pos_v)
        pltpu.sync_copy(x_hbm.at[pl.ds(base, CHUNK)], x_v)
        pltpu.async_copy(x_v, xs_hbm.at[pos_v], sem).wait()

    return _dispatch


@functools.cache
def _get_combine():
    mesh = plsc.VectorSubcoreMesh(core_axis_name="c", subcore_axis_name="s")

    @functools.partial(
        pl.kernel,
        mesh=mesh,
        out_type=jax.ShapeDtypeStruct((T, H), jnp.float32),
        scratch_types=[
            pltpu.VMEM((CHUNK,), jnp.int32),
            pltpu.VMEM((CHUNK, H), jnp.float32),
            pltpu.SemaphoreType.DMA,
        ],
    )
    def _combine(ys_hbm, pos_hbm, out_hbm, pos_v, y_v, sem):
        wid = lax.axis_index("s") * NC + lax.axis_index("c")
        base = wid * CHUNK
        pltpu.sync_copy(pos_hbm.at[pl.ds(base, CHUNK)], pos_v)
        pltpu.async_copy(ys_hbm.at[pos_v], y_v, sem).wait()
        pltpu.sync_copy(y_v, out_hbm.at[pl.ds(base, CHUNK)])

    return _combine


def _ffn_body(tbl_ref, xs_ref, fcw_ref, fcb_ref, outw_ref, outb_ref, ys_ref):
    s = pl.program_id(0)
    e = tbl_ref[1, s]
    b = tbl_ref[0, s]
    x = xs_ref[...].astype(jnp.bfloat16)
    h = lax.dot_general(x, fcw_ref[0].astype(jnp.bfloat16),
                        (((1,), (1,)), ((), ())),
                        preferred_element_type=jnp.float32)
    h = h + fcb_ref[pl.ds(e, 1), :]
    g = h[:, :F_DIM] * _gelu(h[:, F_DIM:])
    eo = lax.dot_general(g.astype(jnp.bfloat16),
                         outw_ref[0].astype(jnp.bfloat16),
                         (((1,), (1,)), ((), ())),
                         preferred_element_type=jnp.float32)
    eo = eo + outb_ref[pl.ds(e, 1), :]
    r = b * BF + lax.broadcasted_iota(jnp.int32, (BF, 1), 0)
    keep = (r >= tbl_ref[4, e]) & (r < tbl_ref[4, e + 1]) & (tbl_ref[3, s] > 0)
    contrib = jnp.where(keep, eo, 0.0)

    @pl.when(tbl_ref[2, s] == 1)
    def _():
        ys_ref[...] = contrib

    @pl.when(tbl_ref[2, s] != 1)
    def _():
        ys_ref[...] = ys_ref[...] + contrib


def _ffn_call(tbl, xs, fc_w, fc_b, out_w, out_b):
    grid_spec = pltpu.PrefetchScalarGridSpec(
        num_scalar_prefetch=1,
        grid=(STEPS,),
        in_specs=[
            pl.BlockSpec((BF, H), lambda s, tbl: (tbl[0, s], 0)),
            pl.BlockSpec((1, F2, H), lambda s, tbl: (tbl[1, s], 0, 0)),
            pl.BlockSpec((E, F2), lambda s, tbl: (0, 0)),
            pl.BlockSpec((1, H, F_DIM), lambda s, tbl: (tbl[1, s], 0, 0)),
            pl.BlockSpec((E, H), lambda s, tbl: (0, 0)),
        ],
        out_specs=pl.BlockSpec((BF, H), lambda s, tbl: (tbl[0, s], 0)),
    )
    return pl.pallas_call(
        _ffn_body,
        grid_spec=grid_spec,
        out_shape=jax.ShapeDtypeStruct((T, H), jnp.float32),
    )(tbl, xs, fc_w, fc_b, out_w, out_b)


def kernel(x, gate_w, gate_b, fc_w, fc_b, out_w, out_b):
    pos2, loss11, tbl = _router_call(x, gate_w, gate_b)
    pos = pos2.reshape(T)
    xs = _get_dispatch()(x, pos)
    ys = _ffn_call(tbl, xs, fc_w, fc_b, out_w, out_b)
    out = _get_combine()(ys, pos)
    return out, loss11.reshape(())


# speedup vs baseline: 1.2211x; 1.0579x over previous
import functools

import jax
import jax.numpy as jnp
from jax import lax
from jax.experimental import pallas as pl
from jax.experimental.pallas import tpu as pltpu
from jax.experimental.pallas import tpu_sc as plsc

H = 768
F_DIM = 768
F2 = 2 * F_DIM
E = 8
T = 2048
BT = 256
NB = T // BT
BF = 256
NF = T // BF
LOG_BF = 8
STEPS = NF + E - 1
TW = 32
NC = 2
NS = 16
NW = NC * NS
CHUNK = T // NW


def _erf(v):
    p = 0.3275911
    a1, a2, a3, a4, a5 = (0.254829592, -0.284496736, 1.421413741,
                          -1.453152027, 1.061405429)
    sg = jnp.sign(v)
    av = jnp.abs(v)
    t = 1.0 / (1.0 + p * av)
    poly = ((((a5 * t + a4) * t + a3) * t + a2) * t + a1) * t
    return sg * (1.0 - poly * jnp.exp(-av * av))


def _gelu(v):
    return 0.5 * v * (1.0 + _erf(v * 0.7071067811865476))


def _incl_scan(v, width):
    def _sh(u, k):
        return jnp.concatenate(
            [jnp.zeros((1, k), jnp.float32), u[:, :width - k]], axis=1)

    k = 1
    while k < width:
        v = v + _sh(v, k)
        k *= 2
    return v


def _lanes_to_sublanes(v):
    d = (lax.broadcasted_iota(jnp.int32, (E, E), 0)
         == lax.broadcasted_iota(jnp.int32, (E, E), 1))
    return jnp.sum(jnp.where(d, v, 0.0), axis=1, keepdims=True)


def _router_body(gate_w_ref, gate_b_ref, x_ref, pos_ref, loss_ref, tbl_ref,
                 idx_scr, rank_scr, cnt_scr):
    s = pl.program_id(0)

    @pl.when(s == 0)
    def _():
        cnt_scr[...] = jnp.zeros_like(cnt_scr)

    @pl.when(s < NB)
    def _():
        x = x_ref[...]
        raw = lax.dot_general(x, gate_w_ref[...], (((1,), (1,)), ((), ())),
                              preferred_element_type=jnp.float32)
        raw = raw + gate_b_ref[...]
        eids = lax.broadcasted_iota(jnp.int32, (BT, E), 1)
        m = jnp.max(raw, axis=1, keepdims=True)
        idx = jnp.min(jnp.where(raw == m, eids, E), axis=1)
        onehot = (eids == idx[:, None]).astype(jnp.bfloat16)

        rit = lax.broadcasted_iota(jnp.int32, (BT, BT), 0)
        cit = lax.broadcasted_iota(jnp.int32, (BT, BT), 1)
        ltri = (rit > cit).astype(jnp.bfloat16)
        within = lax.dot_general(ltri, onehot, (((1,), (0,)), ((), ())),
                                 preferred_element_type=jnp.float32)
        rank_mat = within + cnt_scr[...]
        rank = jnp.sum(jnp.where(eids == idx[:, None], rank_mat, 0.0), axis=1)

        idx_scr[pl.ds(s * BT, BT), :] = idx[:, None]
        rank_scr[pl.ds(s * BT, BT), :] = rank.astype(jnp.int32)[:, None]
        cnt_scr[...] = cnt_scr[...] + jnp.sum(
            onehot.astype(jnp.float32), axis=0, keepdims=True)

    @pl.when(s == NB)
    def _():
        cnt = cnt_scr[...]
        usage = (cnt > 0.0).astype(jnp.float32)
        loss_ref[...] = (jnp.sum((usage - 1.0 / E) ** 2) + 1e-8).reshape(1, 1)

        cw = jnp.concatenate([cnt, jnp.zeros((1, TW - E), jnp.float32)],
                             axis=1)
        incl = _incl_scan(cw, TW)
        offs_row = incl - cw

        idx_all = idx_scr[...]
        rank_all = rank_scr[...]
        eids_t = lax.broadcasted_iota(jnp.int32, (T, E), 1)
        offs_e = offs_row[:, :E]
        picked = jnp.sum(jnp.where(eids_t == idx_all, offs_e, 0.0),
                         axis=1, keepdims=True)
        pos = rank_all + picked.astype(jnp.int32)
        pos_ref[...] = pos.reshape(T // 128, 128)

        lanew = lax.broadcasted_iota(jnp.int32, (1, TW), 1)
        offs_i = offs_row.astype(jnp.int32)
        c_i = cw.astype(jnp.int32)
        blk_start = lax.shift_right_logical(offs_i, LOG_BF)
        blk_end = jnp.where(c_i > 0,
                            lax.shift_right_logical(offs_i + c_i - 1, LOG_BF),
                            blk_start - 1)
        nblk = jnp.maximum(blk_end - blk_start + 1, 0)
        nblk = jnp.where(lanew < E, nblk, 0).astype(jnp.float32)
        cumnb = _incl_scan(nblk, TW)
        cumnb_excl = cumnb - nblk
        tp = jnp.max(cumnb)

        cumnb_sub = _lanes_to_sublanes(cumnb[:, :E])
        base_sub = _lanes_to_sublanes(
            (blk_start.astype(jnp.float32) - cumnb_excl)[:, :E])
        sw = lanew.astype(jnp.float32)
        eid = jnp.sum((cumnb_sub <= sw).astype(jnp.float32),
                      axis=0, keepdims=True)
        eid = jnp.minimum(eid, float(E - 1))
        erow = lax.broadcasted_iota(jnp.int32, (E, TW), 0).astype(jnp.float32)
        base = jnp.sum(jnp.where(erow == eid, base_sub, 0.0),
                       axis=0, keepdims=True)
        valid = (sw < tp).astype(jnp.float32)
        bid = jnp.where(valid > 0, base + sw, float(NF - 1))
        prev = jnp.concatenate(
            [jnp.full((1, 1), -1.0, jnp.float32), bid[:, :TW - 1]], axis=1)
        first = ((lanew == 0) | (bid != prev)).astype(jnp.float32)
        last_eid = jnp.max(jnp.where(valid > 0, eid, -1.0))
        eid = jnp.where(valid > 0, eid, last_eid)

        prev_e = jnp.concatenate([eid[:, :1], eid[:, :TW - 1]], axis=1)
        echg = ((lanew > 0) & (eid != prev_e)).astype(jnp.float32)
        run_idx = _incl_scan(echg, TW)
        has = (nblk > 0).astype(jnp.float32)
        runrank = _incl_scan(has, TW) - has
        has_sub = _lanes_to_sublanes(has[:, :E])
        rrank_sub = _lanes_to_sublanes(runrank[:, :E])
        evals = lax.broadcasted_iota(jnp.int32, (E, TW), 0).astype(jnp.float32)
        run_eid = jnp.sum(
            jnp.where((has_sub > 0) & (rrank_sub == sw), evals, 0.0),
            axis=0, keepdims=True)
        nruns = jnp.max(_incl_scan(has, TW))
        run_valid = (sw < nruns).astype(jnp.float32)

        tbl = jnp.concatenate(
            [bid, eid, first, valid, offs_row, run_idx, run_eid, run_valid],
            axis=0)
        tbl_ref[...] = tbl.astype(jnp.int32)


def _router_call(x, gate_w, gate_b):
    return pl.pallas_call(
        _router_body,
        grid=(NB + 1,),
        in_specs=[
            pl.BlockSpec((E, H), lambda s: (0, 0)),
            pl.BlockSpec((1, E), lambda s: (0, 0)),
            pl.BlockSpec((BT, H), lambda s: (s % NB, 0)),
        ],
        out_specs=[
            pl.BlockSpec((T // 128, 128), lambda s: (0, 0)),
            pl.BlockSpec((1, 1), lambda s: (0, 0)),
            pl.BlockSpec((E, TW), lambda s: (0, 0)),
        ],
        out_shape=[
            jax.ShapeDtypeStruct((T // 128, 128), jnp.int32),
            jax.ShapeDtypeStruct((1, 1), jnp.float32),
            jax.ShapeDtypeStruct((E, TW), jnp.int32),
        ],
        scratch_shapes=[pltpu.VMEM((T, 1), jnp.int32),
                        pltpu.VMEM((T, 1), jnp.int32),
                        pltpu.VMEM((1, E), jnp.float32)],
    )(gate_w, gate_b.reshape(1, E), x)


@functools.cache
def _get_dispatch():
    mesh = plsc.VectorSubcoreMesh(core_axis_name="c", subcore_axis_name="s")

    @functools.partial(
        pl.kernel,
        mesh=mesh,
        out_type=jax.ShapeDtypeStruct((T, H), jnp.float32),
        scratch_types=[
            pltpu.VMEM((CHUNK,), jnp.int32),
            pltpu.VMEM((CHUNK, H), jnp.float32),
            pltpu.SemaphoreType.DMA,
        ],
    )
    def _dispatch(x_hbm, pos_hbm, xs_hbm, pos_v, x_v, sem):
        wid = lax.axis_index("s") * NC + lax.axis_index("c")
        base = wid * CHUNK
        pltpu.sync_copy(pos_hbm.at[pl.ds(base, CHUNK)], pos_v)
        pltpu.sync_copy(x_hbm.at[pl.ds(base, CHUNK)], x_v)
        pltpu.async_copy(x_v, xs_hbm.at[pos_v], sem).wait()

    return _dispatch


@functools.cache
def _get_combine():
    mesh = plsc.VectorSubcoreMesh(core_axis_name="c", subcore_axis_name="s")

    @functools.partial(
        pl.kernel,
        mesh=mesh,
        out_type=jax.ShapeDtypeStruct((T, H), jnp.float32),
        scratch_types=[
            pltpu.VMEM((CHUNK,), jnp.int32),
            pltpu.VMEM((CHUNK, H), jnp.float32),
            pltpu.SemaphoreType.DMA,
        ],
    )
    def _combine(ys_hbm, pos_hbm, out_hbm, pos_v, y_v, sem):
        wid = lax.axis_index("s") * NC + lax.axis_index("c")
        base = wid * CHUNK
        pltpu.sync_copy(pos_hbm.at[pl.ds(base, CHUNK)], pos_v)
        pltpu.async_copy(ys_hbm.at[pos_v], y_v, sem).wait()
        pltpu.sync_copy(y_v, out_hbm.at[pl.ds(base, CHUNK)])

    return _combine


def _ffn_body(tbl_ref, xs_ref, fcw_hbm, fcb_ref, outw_hbm, outb_ref, ys_ref,
              w1_scr, w2_scr, sems):
    s = pl.program_id(0)
    e = tbl_ref[1, s]
    b = tbl_ref[0, s]
    r = tbl_ref[5, s]
    slot = lax.rem(r, 2)
    prev_r = tbl_ref[5, jnp.maximum(s - 1, 0)]
    efirst = jnp.logical_or(s == 0, r != prev_r)

    def _fetch(run):
        re = tbl_ref[6, run]
        sl = lax.rem(run, 2)
        pltpu.make_async_copy(fcw_hbm.at[re], w1_scr.at[sl],
                              sems.at[sl]).start()
        pltpu.make_async_copy(outw_hbm.at[re], w2_scr.at[sl],
                              sems.at[sl]).start()

    @pl.when(s == 0)
    def _():
        _fetch(0)

    @pl.when(efirst & (tbl_ref[7, r + 1] == 1))
    def _():
        _fetch(r + 1)

    @pl.when(efirst)
    def _():
        pltpu.make_async_copy(fcw_hbm.at[e], w1_scr.at[slot],
                              sems.at[slot]).wait()
        pltpu.make_async_copy(outw_hbm.at[e], w2_scr.at[slot],
                              sems.at[slot]).wait()

    x = xs_ref[...].astype(jnp.bfloat16)
    h = lax.dot_general(x, w1_scr[slot].astype(jnp.bfloat16),
                        (((1,), (1,)), ((), ())),
                        preferred_element_type=jnp.float32)
    h = h + fcb_ref[pl.ds(e, 1), :]
    g = h[:, :F_DIM] * _gelu(h[:, F_DIM:])
    eo = lax.dot_general(g.astype(jnp.bfloat16),
                         w2_scr[slot].astype(jnp.bfloat16),
                         (((1,), (1,)), ((), ())),
                         preferred_element_type=jnp.float32)
    eo = eo + outb_ref[pl.ds(e, 1), :]
    rr = b * BF + lax.broadcasted_iota(jnp.int32, (BF, 1), 0)
    keep = ((rr >= tbl_ref[4, e]) & (rr < tbl_ref[4, e + 1])
            & (tbl_ref[3, s] > 0))
    contrib = jnp.where(keep, eo, 0.0)

    @pl.when(tbl_ref[2, s] == 1)
    def _():
        ys_ref[...] = contrib

    @pl.when(tbl_ref[2, s] != 1)
    def _():
        ys_ref[...] = ys_ref[...] + contrib


def _ffn_call(tbl, xs, fc_w, fc_b, out_w, out_b):
    grid_spec = pltpu.PrefetchScalarGridSpec(
        num_scalar_prefetch=1,
        grid=(STEPS,),
        in_specs=[
            pl.BlockSpec((BF, H), lambda s, tbl: (tbl[0, s], 0)),
            pl.BlockSpec(memory_space=pltpu.MemorySpace.HBM),
            pl.BlockSpec((E, F2), lambda s, tbl: (0, 0)),
            pl.BlockSpec(memory_space=pltpu.MemorySpace.HBM),
            pl.BlockSpec((E, H), lambda s, tbl: (0, 0)),
        ],
        out_specs=pl.BlockSpec((BF, H), lambda s, tbl: (tbl[0, s], 0)),
        scratch_shapes=[
            pltpu.VMEM((2, F2, H), jnp.float32),
            pltpu.VMEM((2, H, F_DIM), jnp.float32),
            pltpu.SemaphoreType.DMA((2,)),
        ],
    )
    return pl.pallas_call(
        _ffn_body,
        grid_spec=grid_spec,
        out_shape=jax.ShapeDtypeStruct((T, H), jnp.float32),
    )(tbl, xs, fc_w, fc_b, out_w, out_b)


def kernel(x, gate_w, gate_b, fc_w, fc_b, out_w, out_b):
    pos2, loss11, tbl = _router_call(x, gate_w, gate_b)
    pos = pos2.reshape(T)
    xs = _get_dispatch()(x, pos)
    ys = _ffn_call(tbl, xs, fc_w, fc_b, out_w, out_b)
    out = _get_combine()(ys, pos)
    return out, loss11.reshape(())


# speedup vs baseline: 1.2529x; 1.0260x over previous
import functools

import jax
import jax.numpy as jnp
from jax import lax
from jax.experimental import pallas as pl
from jax.experimental.pallas import tpu as pltpu
from jax.experimental.pallas import tpu_sc as plsc

H = 768
F_DIM = 768
F2 = 2 * F_DIM
E = 8
T = 2048
BT = 512
NB = T // BT
BF = 256
NF = T // BF
LOG_BF = 8
STEPS = NF + E - 1
TW = 32
NC = 2
NS = 16
NW = NC * NS
CHUNK = T // NW


def _erf(v):
    p = 0.3275911
    a1, a2, a3, a4, a5 = (0.254829592, -0.284496736, 1.421413741,
                          -1.453152027, 1.061405429)
    sg = jnp.sign(v)
    av = jnp.abs(v)
    t = 1.0 / (1.0 + p * av)
    poly = ((((a5 * t + a4) * t + a3) * t + a2) * t + a1) * t
    return sg * (1.0 - poly * jnp.exp(-av * av))


def _gelu(v):
    return 0.5 * v * (1.0 + _erf(v * 0.7071067811865476))


def _incl_scan(v, width):
    def _sh(u, k):
        return jnp.concatenate(
            [jnp.zeros((1, k), jnp.float32), u[:, :width - k]], axis=1)

    k = 1
    while k < width:
        v = v + _sh(v, k)
        k *= 2
    return v


def _lanes_to_sublanes(v):
    d = (lax.broadcasted_iota(jnp.int32, (E, E), 0)
         == lax.broadcasted_iota(jnp.int32, (E, E), 1))
    return jnp.sum(jnp.where(d, v, 0.0), axis=1, keepdims=True)


def _router_body(gate_w_ref, gate_b_ref, x_ref, pos_ref, loss_ref, tbl_ref,
                 idx_scr, rank_scr, cnt_scr):
    s = pl.program_id(0)

    @pl.when(s == 0)
    def _():
        cnt_scr[...] = jnp.zeros_like(cnt_scr)

    @pl.when(s < NB)
    def _():
        x = x_ref[...]
        raw = lax.dot_general(x, gate_w_ref[...], (((1,), (1,)), ((), ())),
                              preferred_element_type=jnp.float32)
        raw = raw + gate_b_ref[...]
        eids = lax.broadcasted_iota(jnp.int32, (BT, E), 1)
        m = jnp.max(raw, axis=1, keepdims=True)
        idx = jnp.min(jnp.where(raw == m, eids, E), axis=1)
        onehot = (eids == idx[:, None]).astype(jnp.bfloat16)

        rit = lax.broadcasted_iota(jnp.int32, (BT, BT), 0)
        cit = lax.broadcasted_iota(jnp.int32, (BT, BT), 1)
        ltri = (rit > cit).astype(jnp.bfloat16)
        within = lax.dot_general(ltri, onehot, (((1,), (0,)), ((), ())),
                                 preferred_element_type=jnp.float32)
        rank_mat = within + cnt_scr[...]
        rank = jnp.sum(jnp.where(eids == idx[:, None], rank_mat, 0.0), axis=1)

        idx_scr[pl.ds(s * BT, BT), :] = idx[:, None]
        rank_scr[pl.ds(s * BT, BT), :] = rank.astype(jnp.int32)[:, None]
        cnt_scr[...] = cnt_scr[...] + jnp.sum(
            onehot.astype(jnp.float32), axis=0, keepdims=True)

    @pl.when(s == NB)
    def _():
        cnt = cnt_scr[...]
        usage = (cnt > 0.0).astype(jnp.float32)
        loss_ref[...] = (jnp.sum((usage - 1.0 / E) ** 2) + 1e-8).reshape(1, 1)

        cw = jnp.concatenate([cnt, jnp.zeros((1, TW - E), jnp.float32)],
                             axis=1)
        incl = _incl_scan(cw, TW)
        offs_row = incl - cw

        idx_all = idx_scr[...]
        rank_all = rank_scr[...]
        eids_t = lax.broadcasted_iota(jnp.int32, (T, E), 1)
        offs_e = offs_row[:, :E]
        picked = jnp.sum(jnp.where(eids_t == idx_all, offs_e, 0.0),
                         axis=1, keepdims=True)
        pos = rank_all + picked.astype(jnp.int32)
        pos_ref[...] = pos.reshape(T // 128, 128)

        lanew = lax.broadcasted_iota(jnp.int32, (1, TW), 1)
        offs_i = offs_row.astype(jnp.int32)
        c_i = cw.astype(jnp.int32)
        blk_start = lax.shift_right_logical(offs_i, LOG_BF)
        blk_end = jnp.where(c_i > 0,
                            lax.shift_right_logical(offs_i + c_i - 1, LOG_BF),
                            blk_start - 1)
        nblk = jnp.maximum(blk_end - blk_start + 1, 0)
        nblk = jnp.where(lanew < E, nblk, 0).astype(jnp.float32)
        cumnb = _incl_scan(nblk, TW)
        cumnb_excl = cumnb - nblk
        tp = jnp.max(cumnb)

        cumnb_sub = _lanes_to_sublanes(cumnb[:, :E])
        base_sub = _lanes_to_sublanes(
            (blk_start.astype(jnp.float32) - cumnb_excl)[:, :E])
        sw = lanew.astype(jnp.float32)
        eid = jnp.sum((cumnb_sub <= sw).astype(jnp.float32),
                      axis=0, keepdims=True)
        eid = jnp.minimum(eid, float(E - 1))
        erow = lax.broadcasted_iota(jnp.int32, (E, TW), 0).astype(jnp.float32)
        base = jnp.sum(jnp.where(erow == eid, base_sub, 0.0),
                       axis=0, keepdims=True)
        valid = (sw < tp).astype(jnp.float32)
        bid = jnp.where(valid > 0, base + sw, float(NF - 1))
        prev = jnp.concatenate(
            [jnp.full((1, 1), -1.0, jnp.float32), bid[:, :TW - 1]], axis=1)
        first = ((lanew == 0) | (bid != prev)).astype(jnp.float32)
        last_eid = jnp.max(jnp.where(valid > 0, eid, -1.0))
        eid = jnp.where(valid > 0, eid, last_eid)

        prev_e = jnp.concatenate([eid[:, :1], eid[:, :TW - 1]], axis=1)
        echg = ((lanew > 0) & (eid != prev_e)).astype(jnp.float32)
        run_idx = _incl_scan(echg, TW)
        has = (nblk > 0).astype(jnp.float32)
        runrank = _incl_scan(has, TW) - has
        has_sub = _lanes_to_sublanes(has[:, :E])
        rrank_sub = _lanes_to_sublanes(runrank[:, :E])
        evals = lax.broadcasted_iota(jnp.int32, (E, TW), 0).astype(jnp.float32)
        run_eid = jnp.sum(
            jnp.where((has_sub > 0) & (rrank_sub == sw), evals, 0.0),
            axis=0, keepdims=True)
        nruns = jnp.max(_incl_scan(has, TW))
        run_valid = (sw < nruns).astype(jnp.float32)

        tbl = jnp.concatenate(
            [bid, eid, first, valid, offs_row, run_idx, run_eid, run_valid],
            axis=0)
        tbl_ref[...] = tbl.astype(jnp.int32)


def _router_call(x, gate_w, gate_b):
    return pl.pallas_call(
        _router_body,
        grid=(NB + 1,),
        in_specs=[
            pl.BlockSpec((E, H), lambda s: (0, 0)),
            pl.BlockSpec((1, E), lambda s: (0, 0)),
            pl.BlockSpec((BT, H), lambda s: (s % NB, 0)),
        ],
        out_specs=[
            pl.BlockSpec((T // 128, 128), lambda s: (0, 0)),
            pl.BlockSpec((1, 1), lambda s: (0, 0)),
            pl.BlockSpec((E, TW), lambda s: (0, 0)),
        ],
        out_shape=[
            jax.ShapeDtypeStruct((T // 128, 128), jnp.int32),
            jax.ShapeDtypeStruct((1, 1), jnp.float32),
            jax.ShapeDtypeStruct((E, TW), jnp.int32),
        ],
        scratch_shapes=[pltpu.VMEM((T, 1), jnp.int32),
                        pltpu.VMEM((T, 1), jnp.int32),
                        pltpu.VMEM((1, E), jnp.float32)],
    )(gate_w, gate_b.reshape(1, E), x)


@functools.cache
def _get_dispatch():
    mesh = plsc.VectorSubcoreMesh(core_axis_name="c", subcore_axis_name="s")

    @functools.partial(
        pl.kernel,
        mesh=mesh,
        out_type=jax.ShapeDtypeStruct((T, H), jnp.float32),
        scratch_types=[
            pltpu.VMEM((CHUNK,), jnp.int32),
            pltpu.VMEM((CHUNK, H), jnp.float32),
            pltpu.SemaphoreType.DMA,
        ],
    )
    def _dispatch(x_hbm, pos_hbm, xs_hbm, pos_v, x_v, sem):
        wid = lax.axis_index("s") * NC + lax.axis_index("c")
        base = wid * CHUNK
        pltpu.sync_copy(pos_hbm.at[pl.ds(base, CHUNK)], pos_v)
        pltpu.sync_copy(x_hbm.at[pl.ds(base, CHUNK)], x_v)
        pltpu.async_copy(x_v, xs_hbm.at[pos_v], sem).wait()

    return _dispatch


@functools.cache
def _get_combine():
    mesh = plsc.VectorSubcoreMesh(core_axis_name="c", subcore_axis_name="s")

    @functools.partial(
        pl.kernel,
        mesh=mesh,
        out_type=jax.ShapeDtypeStruct((T, H), jnp.float32),
        scratch_types=[
            pltpu.VMEM((CHUNK,), jnp.int32),
            pltpu.VMEM((CHUNK, H), jnp.float32),
            pltpu.SemaphoreType.DMA,
        ],
    )
    def _combine(ys_hbm, pos_hbm, out_hbm, pos_v, y_v, sem):
        wid = lax.axis_index("s") * NC + lax.axis_index("c")
        base = wid * CHUNK
        pltpu.sync_copy(pos_hbm.at[pl.ds(base, CHUNK)], pos_v)
        pltpu.async_copy(ys_hbm.at[pos_v], y_v, sem).wait()
        pltpu.sync_copy(y_v, out_hbm.at[pl.ds(base, CHUNK)])

    return _combine


def _ffn_body(tbl_ref, xs_ref, fcw_hbm, fcb_ref, outw_hbm, outb_ref, ys_ref,
              w1_scr, w2_scr, sems):
    s = pl.program_id(0)
    e = tbl_ref[1, s]
    b = tbl_ref[0, s]
    r = tbl_ref[5, s]
    slot = lax.rem(r, 2)
    prev_r = tbl_ref[5, jnp.maximum(s - 1, 0)]
    efirst = jnp.logical_or(s == 0, r != prev_r)

    def _fetch(run):
        re = tbl_ref[6, run]
        sl = lax.rem(run, 2)
        pltpu.make_async_copy(fcw_hbm.at[re], w1_scr.at[sl],
                              sems.at[sl]).start()
        pltpu.make_async_copy(outw_hbm.at[re], w2_scr.at[sl],
                              sems.at[sl]).start()

    @pl.when(s == 0)
    def _():
        _fetch(0)

    @pl.when(efirst & (tbl_ref[7, r + 1] == 1))
    def _():
        _fetch(r + 1)

    @pl.when(efirst)
    def _():
        pltpu.make_async_copy(fcw_hbm.at[e], w1_scr.at[slot],
                              sems.at[slot]).wait()
        pltpu.make_async_copy(outw_hbm.at[e], w2_scr.at[slot],
                              sems.at[slot]).wait()

    x = xs_ref[...].astype(jnp.bfloat16)
    h = lax.dot_general(x, w1_scr[slot].astype(jnp.bfloat16),
                        (((1,), (1,)), ((), ())),
                        preferred_element_type=jnp.float32)
    h = h + fcb_ref[pl.ds(e, 1), :]
    g = h[:, :F_DIM] * _gelu(h[:, F_DIM:])
    eo = lax.dot_general(g.astype(jnp.bfloat16),
                         w2_scr[slot].astype(jnp.bfloat16),
                         (((1,), (1,)), ((), ())),
                         preferred_element_type=jnp.float32)
    eo = eo + outb_ref[pl.ds(e, 1), :]
    rr = b * BF + lax.broadcasted_iota(jnp.int32, (BF, 1), 0)
    keep = ((rr >= tbl_ref[4, e]) & (rr < tbl_ref[4, e + 1])
            & (tbl_ref[3, s] > 0))
    contrib = jnp.where(keep, eo, 0.0)

    @pl.when(tbl_ref[2, s] == 1)
    def _():
        ys_ref[...] = contrib

    @pl.when(tbl_ref[2, s] != 1)
    def _():
        ys_ref[...] = ys_ref[...] + contrib


def _ffn_call(tbl, xs, fc_w, fc_b, out_w, out_b):
    grid_spec = pltpu.PrefetchScalarGridSpec(
        num_scalar_prefetch=1,
        grid=(STEPS,),
        in_specs=[
            pl.BlockSpec((BF, H), lambda s, tbl: (tbl[0, s], 0)),
            pl.BlockSpec(memory_space=pltpu.MemorySpace.HBM),
            pl.BlockSpec((E, F2), lambda s, tbl: (0, 0)),
            pl.BlockSpec(memory_space=pltpu.MemorySpace.HBM),
            pl.BlockSpec((E, H), lambda s, tbl: (0, 0)),
        ],
        out_specs=pl.BlockSpec((BF, H), lambda s, tbl: (tbl[0, s], 0)),
        scratch_shapes=[
            pltpu.VMEM((2, F2, H), jnp.float32),
            pltpu.VMEM((2, H, F_DIM), jnp.float32),
            pltpu.SemaphoreType.DMA((2,)),
        ],
    )
    return pl.pallas_call(
        _ffn_body,
        grid_spec=grid_spec,
        out_shape=jax.ShapeDtypeStruct((T, H), jnp.float32),
    )(tbl, xs, fc_w, fc_b, out_w, out_b)


def kernel(x, gate_w, gate_b, fc_w, fc_b, out_w, out_b):
    pos2, loss11, tbl = _router_call(x, gate_w, gate_b)
    pos = pos2.reshape(T)
    xs = _get_dispatch()(x, pos)
    ys = _ffn_call(tbl, xs, fc_w, fc_b, out_w, out_b)
    out = _get_combine()(ys, pos)
    return out, loss11.reshape(())
